# Initial kernel scaffold; baseline (speedup 1.0000x reference)
import jax
import jax.numpy as jnp
from jax.experimental import pallas as pl


def kernel(x, edge_index, edge_type, graph_ids, aux_in, params):
    raise NotImplementedError("write your pallas kernel here")


# speedup vs baseline: 31.4312x; 31.4312x over previous
import functools

import jax
import jax.numpy as jnp
from jax import lax
from jax.experimental import pallas as pl
from jax.experimental.pallas import tpu as pltpu
from jax.experimental.pallas import tpu_sc as plsc

N = 100000
E = 1600000
T = 4
H = 32
HH = 16
D_IN = 128
STEPS = 8
G = 64
AUX = 2

BN = 2000
NB = N // BN

ROWS = 12800
EPAD = ROWS * 128
RPT = ROWS // 16
CH = 8
CHUNKS = RPT // CH
NPAD = 100096
TROWS = NPAD // 16

_f32 = jnp.float32


def _sc_edge_body(alo, ahi, gidx, dstx, zer, olo, ohi,
                  gidx_v, dst_v, rows_v, accum, gsem, ssem):
    cid = lax.axis_index("c")

    def half(acts, out):
        sid = lax.axis_index("s")
        pltpu.sync_copy(zer.at[pl.ds(sid * TROWS, TROWS)],
                        accum.at[pl.ds(sid * TROWS, TROWS)])
        plsc.subcore_barrier()

        def chunk(gi, c):
            row0 = sid * RPT + gi * CH
            pltpu.sync_copy(gidx.at[pl.ds(row0, CH)], gidx_v)
            pltpu.sync_copy(dstx.at[pl.ds(row0, CH)], dst_v)
            gds = [pltpu.async_copy(acts.at[gidx_v.at[b]], rows_v.at[b], gsem)
                   for b in range(CH)]
            for d in gds:
                d.wait()
            sds = [pltpu.async_copy(rows_v.at[b], accum.at[dst_v.at[b]],
                                    ssem, add=True)
                   for b in range(CH)]
            for d in sds:
                d.wait()
            return c

        lax.fori_loop(0, CHUNKS, chunk, 0)
        plsc.subcore_barrier()
        pltpu.sync_copy(accum.at[pl.ds(sid * TROWS, TROWS)],
                        out.at[pl.ds(sid * TROWS, TROWS)])

    @pl.when(cid == 0)
    def _():
        half(alo, olo)

    @pl.when(cid == 1)
    def _():
        half(ahi, ohi)


@functools.cache
def _get_sc_edge():
    mesh = plsc.VectorSubcoreMesh(
        core_axis_name="c", subcore_axis_name="s",
        num_cores=2, num_subcores=16)
    return pl.kernel(
        _sc_edge_body,
        out_type=[jax.ShapeDtypeStruct((NPAD, HH), _f32),
                  jax.ShapeDtypeStruct((NPAD, HH), _f32)],
        mesh=mesh,
        scratch_types=[
            pltpu.VMEM((CH, 128), jnp.int32),
            pltpu.VMEM((CH, 128), jnp.int32),
            pltpu.VMEM((CH, 128, HH), _f32),
            pltpu.VMEM_SHARED((NPAD, HH), _f32),
            pltpu.SemaphoreType.DMA,
            pltpu.SemaphoreType.DMA,
        ],
        compiler_params=pltpu.CompilerParams(use_tc_tiling_on_sc=False),
    )


def _acts_out(h, ew, alo_ref, ahi_ref):
    for t in range(T):
        a = jnp.dot(h, ew[t], preferred_element_type=_f32)
        alo_ref[t] = a[:, :HH]
        ahi_ref[t] = a[:, HH:]


def _embed_body(x_ref, w0, b0, w1, b1, ew_ref, h_ref, alo_ref, ahi_ref):
    h = jax.nn.relu(jnp.dot(x_ref[...], w0[...],
                            preferred_element_type=_f32) + b0[...])
    h = jax.nn.relu(jnp.dot(h, w1[...],
                            preferred_element_type=_f32) + b1[...])
    h_ref[...] = h
    _acts_out(h, ew_ref[...], alo_ref, ahi_ref)


def _full(shape):
    nd = len(shape)
    return pl.BlockSpec(shape, lambda i, _nd=nd: (0,) * _nd)


_embed_call = pl.pallas_call(
    _embed_body,
    grid=(NB,),
    in_specs=[
        pl.BlockSpec((BN, D_IN), lambda i: (i, 0)),
        _full((D_IN, H)), _full((1, H)), _full((H, H)), _full((1, H)),
        _full((T, H, H)),
    ],
    out_specs=[
        pl.BlockSpec((BN, H), lambda i: (i, 0)),
        pl.BlockSpec((T, BN, HH), lambda i: (0, i, 0)),
        pl.BlockSpec((T, BN, HH), lambda i: (0, i, 0)),
    ],
    out_shape=[
        jax.ShapeDtypeStruct((N, H), _f32),
        jax.ShapeDtypeStruct((T, N, HH), _f32),
        jax.ShapeDtypeStruct((T, N, HH), _f32),
    ],
)


def _gru_math(h_ref, alo_ref, ahi_ref, wz, uz, bz, wr, ur, br, wh, uh, bh):
    h = h_ref[...]
    al = alo_ref[...]
    ah = ahi_ref[...]

    def am(w_ref):
        w = w_ref[...]
        return (jnp.dot(al, w[:HH], preferred_element_type=_f32)
                + jnp.dot(ah, w[HH:], preferred_element_type=_f32))

    def hm(v, w_ref):
        return jnp.dot(v, w_ref[...], preferred_element_type=_f32)

    z = jax.nn.sigmoid(am(wz) + hm(h, uz) + bz[...])
    r = jax.nn.sigmoid(am(wr) + hm(h, ur) + br[...])
    hh = jnp.tanh(am(wh) + hm(r * h, uh) + bh[...])
    return (1.0 - z) * h + z * hh


def _gru_acts_body(h_ref, alo_ref, ahi_ref, wz, uz, bz, wr, ur, br,
                   wh, uh, bh, ew_ref, ho_ref, aol_ref, aoh_ref):
    hn = _gru_math(h_ref, alo_ref, ahi_ref, wz, uz, bz, wr, ur, br, wh, uh, bh)
    ho_ref[...] = hn
    _acts_out(hn, ew_ref[...], aol_ref, aoh_ref)


def _gru_last_body(h_ref, alo_ref, ahi_ref, wz, uz, bz, wr, ur, br,
                   wh, uh, bh, ho_ref):
    ho_ref[...] = _gru_math(h_ref, alo_ref, ahi_ref,
                            wz, uz, bz, wr, ur, br, wh, uh, bh)


_gru_in_specs = [
    pl.BlockSpec((BN, H), lambda i: (i, 0)),
    pl.BlockSpec((BN, HH), lambda i: (i, 0)),
    pl.BlockSpec((BN, HH), lambda i: (i, 0)),
] + [_full((H, H)), _full((H, H)), _full((1, H))] * 3

_gru_acts_call = pl.pallas_call(
    _gru_acts_body,
    grid=(NB,),
    in_specs=_gru_in_specs + [_full((T, H, H))],
    out_specs=[
        pl.BlockSpec((BN, H), lambda i: (i, 0)),
        pl.BlockSpec((T, BN, HH), lambda i: (0, i, 0)),
        pl.BlockSpec((T, BN, HH), lambda i: (0, i, 0)),
    ],
    out_shape=[
        jax.ShapeDtypeStruct((N, H), _f32),
        jax.ShapeDtypeStruct((T, N, HH), _f32),
        jax.ShapeDtypeStruct((T, N, HH), _f32),
    ],
)

_gru_last_call = pl.pallas_call(
    _gru_last_body,
    grid=(NB,),
    in_specs=list(_gru_in_specs),
    out_specs=pl.BlockSpec((BN, H), lambda i: (i, 0)),
    out_shape=jax.ShapeDtypeStruct((N, H), _f32),
)


def _mlp3(v, w0, b0, w1, b1, w2, b2):
    v = jax.nn.relu(jnp.dot(v, w0[...], preferred_element_type=_f32) + b0[...])
    v = jax.nn.relu(jnp.dot(v, w1[...], preferred_element_type=_f32) + b1[...])
    return jnp.dot(v, w2[...], preferred_element_type=_f32) + b2[...]


def _readout_body(h_ref, gid_ref, fw0, fb0, fw1, fb1, fw2, fb2,
                  gw0, gb0, gw1, gb1, gw2, gb2, ge_ref):
    h = h_ref[...]
    f = _mlp3(h, fw0, fb0, fw1, fb1, fw2, fb2)
    g = jax.nn.sigmoid(_mlp3(h, gw0, gb0, gw1, gb1, gw2, gb2))
    gated = g * f
    ids = gid_ref[...]
    onehot = (ids == lax.broadcasted_iota(jnp.int32, (BN, G), 1)).astype(_f32)
    part = lax.dot_general(onehot, gated, (((0,), (0,)), ((), ())),
                           preferred_element_type=_f32)

    @pl.when(pl.program_id(0) == 0)
    def _():
        ge_ref[...] = jnp.zeros_like(ge_ref)

    ge_ref[...] += part


_readout_call = pl.pallas_call(
    _readout_body,
    grid=(NB,),
    in_specs=[
        pl.BlockSpec((BN, H), lambda i: (i, 0)),
        pl.BlockSpec((BN, 1), lambda i: (i, 0)),
        _full((H, H)), _full((1, H)), _full((H, H)), _full((1, H)),
        _full((H, G)), _full((1, G)),
        _full((H, H)), _full((1, H)), _full((H, H)), _full((1, H)),
        _full((H, G)), _full((1, G)),
    ],
    out_specs=pl.BlockSpec((G, G), lambda i: (0, 0)),
    out_shape=jax.ShapeDtypeStruct((G, G), _f32),
)


def _final_body(ge_ref, aux_ref, rw0, rb0, rw1, rb1, rw2, rb2,
                a1wa, a1wb, a1b, a2w, a2b, out_ref):
    r1 = _mlp3(ge_ref[...], rw0, rb0, rw1, rb1, rw2, rb2)
    a1 = jax.nn.relu(jnp.dot(r1, a1wa[...], preferred_element_type=_f32)
                     + jnp.dot(aux_ref[...], a1wb[...],
                               preferred_element_type=_f32)
                     + a1b[...])
    out_ref[...] = jax.nn.sigmoid(
        jnp.dot(a1, a2w[...], preferred_element_type=_f32) + a2b[...])


_final_call = pl.pallas_call(
    _final_body,
    out_shape=jax.ShapeDtypeStruct((G, AUX), _f32),
)


def kernel(x, edge_index, edge_type, graph_ids, aux_in, params):
    p = params

    def b2(v):
        return v.reshape(1, -1)

    src = edge_index[0]
    dst = edge_index[1]
    gidx = edge_type * N + src
    padn = EPAD - E
    gidx2 = jnp.concatenate(
        [gidx, jnp.zeros((padn,), jnp.int32)]).reshape(ROWS, 128)
    dst2 = jnp.concatenate(
        [dst, jnp.full((padn,), N, jnp.int32)]).reshape(ROWS, 128)
    zer = jnp.zeros((NPAD, HH), _f32)

    h, alo3, ahi3 = _embed_call(x, p['emb_W0'], b2(p['emb_b0']),
                                p['emb_W1'], b2(p['emb_b1']), p['edge_W'])
    gw = (p['Wz'], p['Uz'], b2(p['bz']), p['Wr'], p['Ur'], b2(p['br']),
          p['Wh'], p['Uh'], b2(p['bh']))
    sc_edge = _get_sc_edge()
    for s in range(STEPS):
        agg_lo, agg_hi = sc_edge(alo3.reshape(T * N, HH),
                                 ahi3.reshape(T * N, HH), gidx2, dst2, zer)
        agg_lo = agg_lo[:N]
        agg_hi = agg_hi[:N]
        if s < STEPS - 1:
            h, alo3, ahi3 = _gru_acts_call(h, agg_lo, agg_hi, *gw,
                                           p['edge_W'])
        else:
            h = _gru_last_call(h, agg_lo, agg_hi, *gw)

    ge = _readout_call(h, graph_ids.reshape(N, 1),
                       p['fm_W0'], b2(p['fm_b0']), p['fm_W1'], b2(p['fm_b1']),
                       p['fm_W2'], b2(p['fm_b2']),
                       p['gm_W0'], b2(p['gm_b0']), p['gm_W1'], b2(p['gm_b1']),
                       p['gm_W2'], b2(p['gm_b2']))
    return _final_call(ge, aux_in,
                       p['red_W0'], b2(p['red_b0']), p['red_W1'],
                       b2(p['red_b1']), p['red_W2'], b2(p['red_b2']),
                       p['aux1_W'][:G], p['aux1_W'][G:], b2(p['aux1_b']),
                       p['aux2_W'], b2(p['aux2_b']))
